# baseline (speedup 1.0000x reference)
import functools

import jax
import jax.numpy as jnp
from jax import lax
from jax.experimental import pallas as pl
from jax.experimental.pallas import tpu as pltpu
from jax.experimental.pallas import tpu_sc as plsc

EMB = 128
ARITY = 6
SHIFTS = tuple(int(k * EMB / ARITY) for k in range(ARITY))
LANES = 16
NVREG = EMB // LANES


@functools.lru_cache(maxsize=None)
def _make_sc_kernel(batch):
    info = plsc.get_sparse_core_info()
    nc, ns = info.num_cores, info.num_subcores
    nw = nc * ns
    bpw = batch // nw
    C = 64
    nchunk = bpw // C
    SUMW = 17

    mesh = plsc.VectorSubcoreMesh(core_axis_name="c", subcore_axis_name="s")

    @functools.partial(
        pl.kernel,
        out_type=jax.ShapeDtypeStruct((batch,), jnp.float32),
        mesh=mesh,
        compiler_params=pltpu.CompilerParams(needs_layout_passes=False),
        scratch_types=[
            pltpu.VMEM((7, nchunk, C), jnp.int32),
            pltpu.VMEM((2, 7, C, EMB), jnp.float32),
            pltpu.VMEM((C * SUMW,), jnp.float32),
            pltpu.VMEM((bpw,), jnp.float32),
            pltpu.SemaphoreType.DMA,
            pltpu.SemaphoreType.DMA,
            pltpu.SemaphoreType.DMA,
        ],
    )
    def sc_kernel(r_h, e1_h, e2_h, e3_h, e4_h, e5_h, e6_h, E_h, R_h,
                  out_h, idx_v, rows_v, sums_v, out_v, sem0, sem1, semi):
        wid = lax.axis_index("s") * nc + lax.axis_index("c")
        base = wid * bpw
        idx_srcs = (r_h, e1_h, e2_h, e3_h, e4_h, e5_h, e6_h)
        stage0 = [pltpu.async_copy(idx_srcs[k].at[pl.ds(base, C)],
                                   idx_v.at[k, 0], semi) for k in range(7)]
        for cp in stage0:
            cp.wait()

        sems = (sem0, sem1)

        def issue(c):
            p = c & 1
            cps = []
            for k in range(7):
                tab = R_h if k == 0 else E_h
                cps.append(pltpu.async_copy(tab.at[idx_v.at[k, c]],
                                            rows_v.at[p, k], sems[p]))
            return cps

        lane = lax.iota(jnp.int32, 16)
        cols = []
        for k in range(7):
            sh = 0 if k == 0 else SHIFTS[k - 1]
            cols.append([(lane + (LANES * i + sh)) & (EMB - 1)
                         for i in range(NVREG)])
        tsum_base = lane * SUMW

        def compute(c):
            p = c & 1

            p_splat = jnp.full((16,), p, jnp.int32)
            k_splats = [jnp.full((16,), k, jnp.int32) for k in range(7)]

            def one_row(r, r_splat):
                refs = [rows_v.at[p, k, r] for k in range(7)]
                prod = [refs[0][pl.ds(LANES * i, LANES)]
                        * refs[1][pl.ds(LANES * i, LANES)]
                        for i in range(NVREG)]
                for k in range(2, 7):
                    sh = SHIFTS[k - 1]
                    for i in range(NVREG):
                        lo = LANES * i + sh
                        if (lo % EMB) + LANES <= EMB:
                            v = refs[k][pl.ds(lo % EMB, LANES)]
                        else:
                            v = plsc.load_gather(
                                rows_v,
                                [p_splat, k_splats[k], r_splat, cols[k][i]])
                        prod[i] = prod[i] * v
                s01 = prod[0] + prod[1]
                s23 = prod[2] + prod[3]
                s45 = prod[4] + prod[5]
                s67 = prod[6] + prod[7]
                sums_v[pl.ds(r * SUMW, LANES)] = (s01 + s23) + (s45 + s67)

            def row_body(it, carry):
                r0 = it * 2
                one_row(r0, jnp.broadcast_to(r0, (16,)))
                one_row(r0 + 1, jnp.broadcast_to(r0 + 1, (16,)))
                return carry

            lax.fori_loop(0, C // 2, row_body, 0)

            for g in range(C // LANES):
                acc = None
                for j in range(LANES):
                    col = plsc.load_gather(
                        sums_v, [tsum_base + (g * LANES * SUMW + j)])
                    acc = col if acc is None else acc + col
                out_v[pl.ds(c * C + g * LANES, LANES)] = acc

        pending = issue(0)
        stage_rest = []
        for c in range(1, nchunk):
            for k in range(7):
                stage_rest.append(pltpu.async_copy(
                    idx_srcs[k].at[pl.ds(base + c * C, C)],
                    idx_v.at[k, c], semi))
        for c in range(nchunk):
            for cp in pending:
                cp.wait()
            if c == 0:
                for cp in stage_rest:
                    cp.wait()
            if c + 1 < nchunk:
                nxt = issue(c + 1)
            compute(c)
            if c + 1 < nchunk:
                pending = nxt

        pltpu.sync_copy(out_v, out_h.at[pl.ds(base, bpw)])

    return sc_kernel


def kernel(r_idx, e1_idx, e2_idx, e3_idx, e4_idx, e5_idx, e6_idx, E, R):
    batch = r_idx.shape[0]
    f = _make_sc_kernel(batch)
    idxs = [jnp.asarray(a, jnp.int32)
            for a in (r_idx, e1_idx, e2_idx, e3_idx, e4_idx, e5_idx, e6_idx)]
    return f(*idxs, E, R)


# speedup vs baseline: 1.0203x; 1.0203x over previous
import functools

import jax
import jax.numpy as jnp
from jax import lax
from jax.experimental import pallas as pl
from jax.experimental.pallas import tpu as pltpu
from jax.experimental.pallas import tpu_sc as plsc

EMB = 128
ARITY = 6
SHIFTS = tuple(int(k * EMB / ARITY) for k in range(ARITY))
LANES = 16
NVREG = EMB // LANES


@functools.lru_cache(maxsize=None)
def _make_sc_kernel(batch):
    info = plsc.get_sparse_core_info()
    nc, ns = info.num_cores, info.num_subcores
    nw = nc * ns
    bpw = batch // nw
    C = 64
    nchunk = bpw // C
    SUMW = 17

    mesh = plsc.VectorSubcoreMesh(core_axis_name="c", subcore_axis_name="s")

    @functools.partial(
        pl.kernel,
        out_type=jax.ShapeDtypeStruct((batch,), jnp.float32),
        mesh=mesh,
        compiler_params=pltpu.CompilerParams(needs_layout_passes=False),
        scratch_types=[
            pltpu.VMEM((7, nchunk, C), jnp.int32),
            pltpu.VMEM((nchunk, 3, 2 * C), jnp.int32),
            pltpu.VMEM((2, C, EMB), jnp.float32),
            pltpu.VMEM((2, 3, 2 * C, EMB), jnp.float32),
            pltpu.VMEM((C * SUMW,), jnp.float32),
            pltpu.VMEM((bpw,), jnp.float32),
            pltpu.SemaphoreType.DMA,
            pltpu.SemaphoreType.DMA,
            pltpu.SemaphoreType.DMA,
        ],
    )
    def sc_kernel(r_h, e1_h, e2_h, e3_h, e4_h, e5_h, e6_h, E_h, R_h,
                  out_h, idx_v, eidx_v, rrows_v, erows_v, sums_v, out_v,
                  sem0, sem1, semi):
        wid = lax.axis_index("s") * nc + lax.axis_index("c")
        base = wid * bpw
        idx_srcs = (r_h, e1_h, e2_h, e3_h, e4_h, e5_h, e6_h)
        def stage_chunk(c):
            cps = [pltpu.async_copy(idx_srcs[0].at[pl.ds(base + c * C, C)],
                                    idx_v.at[0, c], semi)]
            for k in range(1, 7):
                g, half = (k - 1) // 2, ((k - 1) % 2) * C
                cps.append(pltpu.async_copy(
                    idx_srcs[k].at[pl.ds(base + c * C, C)],
                    eidx_v.at[c, g, pl.ds(half, C)], semi))
            return cps

        stage0 = stage_chunk(0)
        for cp in stage0:
            cp.wait()

        sems = (sem0, sem1)

        def issue(c):
            p = c & 1
            cps = [pltpu.async_copy(R_h.at[idx_v.at[0, c]],
                                    rrows_v.at[p], sems[p])]
            for g in range(3):
                cps.append(pltpu.async_copy(E_h.at[eidx_v.at[c, g]],
                                            erows_v.at[p, g], sems[p]))
            return cps

        lane = lax.iota(jnp.int32, 16)
        cols = []
        for k in range(7):
            sh = 0 if k == 0 else SHIFTS[k - 1]
            cols.append([(lane + (LANES * i + sh)) & (EMB - 1)
                         for i in range(NVREG)])
        tsum_base = lane * SUMW

        def compute(c):
            p = c & 1
            p_splat = jnp.full((16,), p, jnp.int32)
            g_splats = [jnp.full((16,), g, jnp.int32) for g in range(3)]

            def one_row(r):
                prod = [rrows_v[p, r, pl.ds(LANES * i, LANES)]
                        * erows_v[p, 0, r, pl.ds(LANES * i, LANES)]
                        for i in range(NVREG)]
                for k in range(2, 7):
                    g, half = (k - 1) // 2, ((k - 1) % 2) * C
                    sh = SHIFTS[k - 1]
                    row = r + half if half else r
                    row_splat = None
                    for i in range(NVREG):
                        lo = LANES * i + sh
                        if (lo % EMB) + LANES <= EMB:
                            v = erows_v[p, g, row, pl.ds(lo % EMB, LANES)]
                        else:
                            if row_splat is None:
                                row_splat = jnp.broadcast_to(row, (16,))
                            v = plsc.load_gather(
                                erows_v,
                                [p_splat, g_splats[g], row_splat, cols[k][i]])
                        prod[i] = prod[i] * v
                s01 = prod[0] + prod[1]
                s23 = prod[2] + prod[3]
                s45 = prod[4] + prod[5]
                s67 = prod[6] + prod[7]
                sums_v[pl.ds(r * SUMW, LANES)] = (s01 + s23) + (s45 + s67)

            def row_body(it, carry):
                one_row(it * 2)
                one_row(it * 2 + 1)
                return carry

            lax.fori_loop(0, C // 2, row_body, 0)

            for g in range(C // LANES):
                acc = None
                for j in range(LANES):
                    col = plsc.load_gather(
                        sums_v, [tsum_base + (g * LANES * SUMW + j)])
                    acc = col if acc is None else acc + col
                out_v[pl.ds(c * C + g * LANES, LANES)] = acc

        pending = issue(0)
        stage_rest = []
        for c in range(1, nchunk):
            stage_rest.extend(stage_chunk(c))
        for c in range(nchunk):
            for cp in pending:
                cp.wait()
            if c == 0:
                for cp in stage_rest:
                    cp.wait()
            if c + 1 < nchunk:
                nxt = issue(c + 1)
            compute(c)
            if c + 1 < nchunk:
                pending = nxt

        pltpu.sync_copy(out_v, out_h.at[pl.ds(base, bpw)])

    return sc_kernel


def kernel(r_idx, e1_idx, e2_idx, e3_idx, e4_idx, e5_idx, e6_idx, E, R):
    batch = r_idx.shape[0]
    f = _make_sc_kernel(batch)
    idxs = [jnp.asarray(a, jnp.int32)
            for a in (r_idx, e1_idx, e2_idx, e3_idx, e4_idx, e5_idx, e6_idx)]
    return f(*idxs, E, R)
